# baseline (speedup 1.0000x reference)
import functools

import jax
import jax.numpy as jnp
from jax import lax
from jax.experimental import pallas as pl
from jax.experimental.pallas import tpu as pltpu
from jax.experimental.pallas import tpu_sc as plsc

NUM_CORES = 2
NUM_SUBCORES = 16
NUM_WORKERS = NUM_CORES * NUM_SUBCORES
LANES = 16
BB = 128
NT = 4
SUP = BB * NT


def kernel(indices, table):
    batch, n_fields = indices.shape
    n_rows, dim = table.shape
    dblk = dim // 8
    bblocks = batch // BB
    assert bblocks == NT * NUM_WORKERS and n_fields % 2 == 0

    idx3 = indices.T.reshape(n_fields, NUM_WORKERS, SUP)
    mesh = plsc.VectorSubcoreMesh(core_axis_name="c", subcore_axis_name="s")

    @functools.partial(
        pl.kernel,
        mesh=mesh,
        out_type=jax.ShapeDtypeStruct((n_fields, dblk, bblocks, 8, BB),
                                      jnp.float32),
        scratch_types=[
            pltpu.VMEM((n_fields, SUP), jnp.int32),
            pltpu.VMEM((SUP, dim), jnp.float32),
            pltpu.VMEM((SUP, dim), jnp.float32),
            pltpu.VMEM((dblk, NT, 8, BB), jnp.float32),
            pltpu.VMEM((dblk, NT, 8, BB), jnp.float32),
            pltpu.SemaphoreType.DMA,
            pltpu.SemaphoreType.DMA,
            pltpu.SemaphoreType.DMA,
            pltpu.SemaphoreType.DMA,
        ],
        compiler_params=pltpu.CompilerParams(
            use_tc_tiling_on_sc=False, needs_layout_passes=False
        ),
    )
    def gather_kernel(idx_hbm, table_hbm, out_hbm, idx_v, gbuf0, gbuf1,
                      sbuf0, sbuf1, gsem0, gsem1, ssem0, ssem1):
        wid = lax.axis_index("s") * NUM_CORES + lax.axis_index("c")
        for f in range(n_fields):
            pltpu.sync_copy(idx_hbm.at[f, wid], idx_v.at[f])

        lane = lax.iota(jnp.int32, LANES)
        row_vecs = [lane + r * LANES for r in range(SUP // LANES)]

        def fire_gather(f, gbuf, gsem):
            pltpu.async_copy(table_hbm.at[idx_v.at[f]], gbuf, gsem)

        def drain_gather(gbuf, gsem):
            pltpu.make_async_copy(
                table_hbm.at[pl.ds(0, SUP)], gbuf, gsem
            ).wait()

        def transpose(gbuf, sbuf):
            for d in range(dim):
                col = jnp.full((LANES,), d, jnp.int32)
                for j in range(NT):
                    vals = [
                        plsc.load_gather(
                            gbuf, [row_vecs[j * (BB // LANES) + h], col]
                        )
                        for h in range(BB // LANES)
                    ]
                    for h, v in enumerate(vals):
                        sbuf[d // 8, j, d % 8, pl.ds(h * LANES, LANES)] = v

        def stores(f, sbuf, ssem, fire):
            for db in range(dblk):
                c = pltpu.make_async_copy(
                    sbuf.at[db], out_hbm.at[f, db, pl.ds(wid * NT, NT)], ssem
                )
                if fire:
                    c.start()
                else:
                    c.wait()

        fire_gather(0, gbuf0, gsem0)

        @pl.loop(0, n_fields, step=2)
        def _(i2):
            @pl.when(i2 > 0)
            def _():
                stores(i2 - 1, sbuf1, ssem1, fire=False)

            fire_gather(i2 + 1, gbuf1, gsem1)
            drain_gather(gbuf0, gsem0)

            @pl.when(i2 > 0)
            def _():
                stores(i2 - 2, sbuf0, ssem0, fire=False)

            transpose(gbuf0, sbuf0)
            stores(i2, sbuf0, ssem0, fire=True)

            @pl.when(i2 + 2 < n_fields)
            def _():
                fire_gather(i2 + 2, gbuf0, gsem0)

            drain_gather(gbuf1, gsem1)
            transpose(gbuf1, sbuf1)
            stores(i2 + 1, sbuf1, ssem1, fire=True)

        stores(n_fields - 2, sbuf0, ssem0, fire=False)
        stores(n_fields - 1, sbuf1, ssem1, fire=False)

    out5 = gather_kernel(idx3, table)
    return out5.transpose(2, 4, 0, 1, 3).reshape(batch, n_fields, dim)


# speedup vs baseline: 1.0184x; 1.0184x over previous
import functools

import jax
import jax.numpy as jnp
from jax import lax
from jax.experimental import pallas as pl
from jax.experimental.pallas import tpu as pltpu
from jax.experimental.pallas import tpu_sc as plsc

NUM_CORES = 2
NUM_SUBCORES = 16
NUM_WORKERS = NUM_CORES * NUM_SUBCORES
LANES = 16
BB = 128


def kernel(indices, table):
    batch, n_fields = indices.shape
    n_rows, dim = table.shape
    total = batch * n_fields
    n_tasks = total // BB
    per_worker = n_tasks // NUM_WORKERS
    assert n_tasks % NUM_WORKERS == 0 and per_worker % 2 == 0
    dblk = dim // 8
    bblocks = batch // BB

    idx3 = indices.T.reshape(NUM_WORKERS, per_worker, BB)
    mesh = plsc.VectorSubcoreMesh(core_axis_name="c", subcore_axis_name="s")

    @functools.partial(
        pl.kernel,
        mesh=mesh,
        out_type=jax.ShapeDtypeStruct((n_fields, dblk, bblocks, 8, BB),
                                      jnp.float32),
        scratch_types=[
            pltpu.VMEM((per_worker, BB), jnp.int32),
            pltpu.VMEM((BB, dim), jnp.float32),
            pltpu.VMEM((BB, dim), jnp.float32),
            pltpu.VMEM((dblk, 8, BB), jnp.float32),
            pltpu.VMEM((dblk, 8, BB), jnp.float32),
            pltpu.SemaphoreType.DMA,
            pltpu.SemaphoreType.DMA,
            pltpu.SemaphoreType.DMA,
            pltpu.SemaphoreType.DMA,
        ],
        compiler_params=pltpu.CompilerParams(
            use_tc_tiling_on_sc=False, needs_layout_passes=False
        ),
    )
    def gather_kernel(idx_hbm, table_hbm, out_hbm, idx_v, gbuf0, gbuf1,
                      sbuf0, sbuf1, gsem0, gsem1, ssem0, ssem1):
        wid = lax.axis_index("s") * NUM_CORES + lax.axis_index("c")
        task0 = wid * per_worker
        pltpu.sync_copy(idx_hbm.at[wid], idx_v)

        lane = lax.iota(jnp.int32, LANES)

        def fire_gather(i, gbuf, gsem):
            pltpu.async_copy(table_hbm.at[idx_v.at[i]], gbuf, gsem)

        def drain_gather(gbuf, gsem):
            pltpu.make_async_copy(
                table_hbm.at[pl.ds(0, BB)], gbuf, gsem
            ).wait()

        row_vecs = [lane + h * LANES for h in range(BB // LANES)]

        def transpose(gbuf, sbuf):
            for d in range(dim):
                col = jnp.full((LANES,), d, jnp.int32)
                vals = [plsc.load_gather(gbuf, [rows, col])
                        for rows in row_vecs]
                for h, v in enumerate(vals):
                    sbuf[d // 8, d % 8, pl.ds(h * LANES, LANES)] = v

        def stores(i, sbuf, ssem, fire):
            t = task0 + i
            f = t // bblocks
            b = t % bblocks
            for db in range(dblk):
                c = pltpu.make_async_copy(
                    sbuf.at[db], out_hbm.at[f, db, b], ssem
                )
                if fire:
                    c.start()
                else:
                    c.wait()

        fire_gather(0, gbuf0, gsem0)

        @pl.loop(0, per_worker, step=2)
        def _(i2):
            @pl.when(i2 > 0)
            def _():
                stores(i2 - 1, sbuf1, ssem1, fire=False)

            fire_gather(i2 + 1, gbuf1, gsem1)
            drain_gather(gbuf0, gsem0)

            @pl.when(i2 > 0)
            def _():
                stores(i2 - 2, sbuf0, ssem0, fire=False)

            transpose(gbuf0, sbuf0)
            stores(i2, sbuf0, ssem0, fire=True)

            @pl.when(i2 + 2 < per_worker)
            def _():
                fire_gather(i2 + 2, gbuf0, gsem0)

            drain_gather(gbuf1, gsem1)
            transpose(gbuf1, sbuf1)
            stores(i2 + 1, sbuf1, ssem1, fire=True)

        stores(per_worker - 2, sbuf0, ssem0, fire=False)
        stores(per_worker - 1, sbuf1, ssem1, fire=False)

    out5 = gather_kernel(idx3, table)
    return out5.transpose(2, 4, 0, 1, 3).reshape(batch, n_fields, dim)


# speedup vs baseline: 1.2228x; 1.2006x over previous
import functools

import jax
import jax.numpy as jnp
from jax import lax
from jax.experimental import pallas as pl
from jax.experimental.pallas import tpu as pltpu
from jax.experimental.pallas import tpu_sc as plsc

NUM_CORES = 2
NUM_SUBCORES = 16
NUM_WORKERS = NUM_CORES * NUM_SUBCORES
LANES = 16
BB = 128


def kernel(indices, table):
    batch, n_fields = indices.shape
    n_rows, dim = table.shape
    total = batch * n_fields
    n_tasks = total // BB
    per_worker = n_tasks // NUM_WORKERS
    assert n_tasks % NUM_WORKERS == 0 and per_worker % 2 == 0
    dblk = dim // 8
    bblocks = batch // BB

    idx3 = indices.T.reshape(NUM_WORKERS, per_worker, BB)
    mesh = plsc.VectorSubcoreMesh(core_axis_name="c", subcore_axis_name="s")

    @functools.partial(
        pl.kernel,
        mesh=mesh,
        out_type=jax.ShapeDtypeStruct((n_fields, dblk, bblocks, 8, BB),
                                      jnp.float32),
        scratch_types=[
            pltpu.VMEM((per_worker, BB), jnp.int32),
            pltpu.VMEM((BB, dim), jnp.float32),
            pltpu.VMEM((BB, dim), jnp.float32),
            pltpu.VMEM((dim, BB), jnp.float32),
            pltpu.VMEM((dim, BB), jnp.float32),
            pltpu.SemaphoreType.DMA,
            pltpu.SemaphoreType.DMA,
            pltpu.SemaphoreType.DMA,
            pltpu.SemaphoreType.DMA,
        ],
        compiler_params=pltpu.CompilerParams(
            use_tc_tiling_on_sc=False, needs_layout_passes=False
        ),
    )
    def gather_kernel(idx_hbm, table_hbm, out_hbm, idx_v, gbuf0, gbuf1,
                      sbuf0, sbuf1, gsem0, gsem1, ssem0, ssem1):
        wid = lax.axis_index("s") * NUM_CORES + lax.axis_index("c")
        task0 = wid * per_worker
        pltpu.sync_copy(idx_hbm.at[wid], idx_v)

        lane = lax.iota(jnp.int32, LANES)

        def fire_gather(i, gbuf, gsem):
            pltpu.async_copy(table_hbm.at[idx_v.at[i]], gbuf, gsem)

        def drain_gather(gbuf, gsem):
            pltpu.make_async_copy(
                table_hbm.at[pl.ds(0, BB)], gbuf, gsem
            ).wait()

        row_vecs = [lane + h * LANES for h in range(BB // LANES)]
        diag = [[((lane + k) & (LANES - 1)) + c0 for k in range(LANES)]
                for c0 in range(0, dim, LANES)]

        def transpose(gbuf, sbuf):
            for rows in row_vecs:
                for cols in diag:
                    vals = [plsc.load_gather(gbuf, [rows, c]) for c in cols]
                    for c, v in zip(cols, vals):
                        plsc.store_scatter(sbuf, [c, rows], v)

        def stores(i, sbuf, ssem, fire):
            t = task0 + i
            f = t // bblocks
            b = t % bblocks
            for db in range(dblk):
                c = pltpu.make_async_copy(
                    sbuf.at[pl.ds(db * 8, 8)], out_hbm.at[f, db, b], ssem
                )
                if fire:
                    c.start()
                else:
                    c.wait()

        fire_gather(0, gbuf0, gsem0)

        @pl.loop(0, per_worker, step=2)
        def _(i2):
            @pl.when(i2 > 0)
            def _():
                stores(i2 - 1, sbuf1, ssem1, fire=False)

            fire_gather(i2 + 1, gbuf1, gsem1)
            drain_gather(gbuf0, gsem0)

            @pl.when(i2 > 0)
            def _():
                stores(i2 - 2, sbuf0, ssem0, fire=False)

            transpose(gbuf0, sbuf0)
            stores(i2, sbuf0, ssem0, fire=True)

            @pl.when(i2 + 2 < per_worker)
            def _():
                fire_gather(i2 + 2, gbuf0, gsem0)

            drain_gather(gbuf1, gsem1)
            transpose(gbuf1, sbuf1)
            stores(i2 + 1, sbuf1, ssem1, fire=True)

        stores(per_worker - 2, sbuf0, ssem0, fire=False)
        stores(per_worker - 1, sbuf1, ssem1, fire=False)

    out5 = gather_kernel(idx3, table)
    return out5.transpose(2, 4, 0, 1, 3).reshape(batch, n_fields, dim)
